# baseline (speedup 1.0000x reference)
import functools

import jax
import jax.numpy as jnp
from jax import lax
from jax.experimental import pallas as pl
from jax.experimental.pallas import tpu as pltpu
from jax.experimental.pallas import tpu_sc as plsc

_VG_W, _VG_H, _VG_T = 32, 24, 5
_XY_SCALE = 0.05
_NBINS = 2 * _VG_T * _VG_H * _VG_W
_NLANE = 16
_NTILES = 32

_N = 4_000_000
_NROWS = _N // 128

_HROWS = 96
_HFULL = 960 // _HROWS

_MROWS = 504
_MCH = 2

_COPIES = 4
_SKEW = _NBINS + 1
_HALLOC = _COPIES * _SKEW

_SC_PARAMS = pltpu.CompilerParams(
    needs_layout_passes=False, use_tc_tiling_on_sc=False)
_sc_mesh = plsc.VectorSubcoreMesh(core_axis_name="c", subcore_axis_name="s")


def _dma_start(src, dst, sem):
    pltpu.make_async_copy(src, dst, sem).start()


def _dma_wait(src, dst, sem):
    pltpu.make_async_copy(src, dst, sem).wait()


@functools.partial(
    pl.kernel,
    mesh=_sc_mesh,
    compiler_params=_SC_PARAMS,
    out_type=(
        jax.ShapeDtypeStruct((_NTILES, 16), jnp.float32),
        jax.ShapeDtypeStruct((_NTILES, 16), jnp.float32),
    ),
    scratch_types=[
        pltpu.VMEM((_MROWS, 1, 128), jnp.float32),
        pltpu.VMEM((_MROWS, 1, 128), jnp.float32),
        pltpu.VMEM((16,), jnp.float32),
        pltpu.VMEM((16,), jnp.float32),
        pltpu.SemaphoreType.DMA,
        pltpu.SemaphoreType.DMA,
    ],
)
def _sc_minmax(ev_hbm, min_hbm, max_hbm, bufa, bufb, minv, maxv, sema, semb):
    wid = lax.axis_index("s") * 2 + lax.axis_index("c")
    s = 976 * wid

    def _st(c):
        return jnp.minimum(s + c * _MROWS, _NROWS - _MROWS)

    _dma_start(ev_hbm.at[pl.ds(_st(0), _MROWS), pl.ds(2, 1), :], bufa, sema)
    _dma_start(ev_hbm.at[pl.ds(_st(1), _MROWS), pl.ds(2, 1), :], bufb, semb)

    def _scan(buf, mn, mx):
        def rbody(j, c):
            mn, mx = c
            for a in range(8):
                t = buf[j, 0, pl.ds(a * 16, 16)]
                mn = jnp.minimum(mn, t)
                mx = jnp.maximum(mx, t)
            return mn, mx

        return lax.fori_loop(0, _MROWS, rbody, (mn, mx))

    def _main(i, c):
        mn, mx = c
        c0 = 2 * i
        _dma_wait(ev_hbm.at[pl.ds(0, _MROWS), pl.ds(2, 1), :], bufa, sema)
        mn, mx = _scan(bufa, mn, mx)

        @pl.when(c0 + 2 < _MCH)
        def _():
            _dma_start(ev_hbm.at[pl.ds(_st(c0 + 2), _MROWS), pl.ds(2, 1), :],
                       bufa, sema)

        _dma_wait(ev_hbm.at[pl.ds(0, _MROWS), pl.ds(2, 1), :], bufb, semb)
        mn, mx = _scan(bufb, mn, mx)

        @pl.when(c0 + 3 < _MCH)
        def _():
            _dma_start(ev_hbm.at[pl.ds(_st(c0 + 3), _MROWS), pl.ds(2, 1), :],
                       bufb, semb)

        return mn, mx

    inf = jnp.full((16,), jnp.inf, jnp.float32)
    mn, mx = lax.fori_loop(0, _MCH // 2, _main, (inf, -inf))
    minv[...] = mn
    maxv[...] = mx
    pltpu.sync_copy(minv, min_hbm.at[wid])
    pltpu.sync_copy(maxv, max_hbm.at[wid])


@functools.partial(
    pl.kernel,
    mesh=_sc_mesh,
    compiler_params=_SC_PARAMS,
    out_type=jax.ShapeDtypeStruct((_NTILES, _NBINS), jnp.float32),
    scratch_types=[
        pltpu.VMEM((_HALLOC,), jnp.float32),
        pltpu.VMEM((_HROWS, 4, 128), jnp.float32),
        pltpu.VMEM((_HROWS, 4, 128), jnp.float32),
        pltpu.VMEM((_NTILES, 16), jnp.float32),
        pltpu.VMEM((_NTILES, 16), jnp.float32),
        pltpu.SemaphoreType.DMA,
        pltpu.SemaphoreType.DMA,
    ],
)
def _sc_hist(ev_hbm, min_hbm, max_hbm, out_hbm,
             hist, bufa, bufb, mnv, mxv, sema, semb):
    wid = lax.axis_index("s") * 2 + lax.axis_index("c")
    s = 976 * wid + jnp.minimum(wid, 18)
    n = jnp.where(wid < 18, 977, 976)

    ii = lax.iota(jnp.int32, 16)
    z16 = jnp.zeros((16,), jnp.float32)
    ones = jnp.ones((16,), jnp.float32)
    lane_offs = [((ii + a) & (_COPIES - 1)) * _SKEW for a in range(8)]

    _dma_start(ev_hbm.at[pl.ds(s, _HROWS), :, :], bufa, sema)
    _dma_start(ev_hbm.at[pl.ds(s + _HROWS, _HROWS), :, :], bufb, semb)

    pltpu.sync_copy(min_hbm, mnv)
    pltpu.sync_copy(max_hbm, mxv)
    mn = mnv[0, :]
    mx = mxv[0, :]
    for r in range(1, _NTILES):
        mn = jnp.minimum(mn, mnv[r, :])
        mx = jnp.maximum(mx, mxv[r, :])
    tmin = jnp.full((16,), jnp.min(mn), jnp.float32)
    tmax = jnp.full((16,), jnp.max(mx), jnp.float32)
    condv = tmax > tmin
    denom = jnp.where(condv, tmax - tmin, 1.0)
    tscl = jnp.where(condv, jnp.float32(_VG_T) / denom, jnp.float32(0.1))
    toff = jnp.where(condv, tmin, 0.0)

    def _zero(i, c):
        b = i * 256
        for k in range(16):
            hist[pl.ds(b + k * 16, 16)] = z16
        return c

    lax.fori_loop(0, _HALLOC // 256, _zero, 0)
    hist[pl.ds(_HALLOC - 16, 16)] = z16

    def _bins(buf, j, a):
        sl = pl.ds(a * 16, 16)
        x = buf[j, 0, sl]
        y = buf[j, 1, sl]
        t = buf[j, 2, sl]
        p = buf[j, 3, sl]
        xv = jnp.clip((x * _XY_SCALE).astype(jnp.int32), 0, _VG_W - 1)
        yv = jnp.clip((y * _XY_SCALE).astype(jnp.int32), 0, _VG_H - 1)
        tv = jnp.clip(((t - toff) * tscl).astype(jnp.int32), 0, _VG_T - 1)
        ch = jnp.where(p > 0.0, 0, _NBINS // 2)
        return ch + tv * (_VG_H * _VG_W) + yv * _VG_W + xv + lane_offs[a]

    def _proc(buf):
        @plsc.parallel_loop(0, _HROWS, unroll=2)
        def rbody(j):
            bs = [_bins(buf, j, a) for a in range(8)]
            for b in bs:
                plsc.addupdate_scatter(hist, [b], ones)

    _TAILW = 977 - _HFULL * _HROWS

    def _main(i, c):
        c0 = 2 * i
        _dma_wait(ev_hbm.at[pl.ds(0, _HROWS), :, :], bufa, sema)
        _proc(bufa)

        @pl.when(c0 + 2 < _HFULL)
        def _():
            _dma_start(ev_hbm.at[pl.ds(s + (c0 + 2) * _HROWS, _HROWS), :, :],
                       bufa, sema)

        _dma_wait(ev_hbm.at[pl.ds(0, _HROWS), :, :], bufb, semb)
        _proc(bufb)

        @pl.when(c0 + 3 < _HFULL)
        def _():
            _dma_start(ev_hbm.at[pl.ds(s + (c0 + 3) * _HROWS, _HROWS), :, :],
                       bufb, semb)

        return c

    lax.fori_loop(0, _HFULL // 2, _main, 0)

    pltpu.sync_copy(ev_hbm.at[pl.ds(s + n - _TAILW, _TAILW), :, :],
                    bufa.at[pl.ds(0, _TAILW), :, :])

    def tbody(r, c):
        keep = (jnp.full((16,), r, jnp.int32) + n) >= 977
        for a in range(8):
            plsc.addupdate_scatter(hist, [_bins(bufa, r, a)], ones, mask=keep)
        return c

    lax.fori_loop(0, _TAILW, tbody, 0)

    def _reduce(j, c):
        o = j * 16
        acc = hist[pl.ds(o, 16)]
        for l in range(1, _COPIES):
            acc = acc + hist[pl.ds(l * _SKEW + o, 16)]
        hist[pl.ds(o, 16)] = acc
        return c

    lax.fori_loop(0, _NBINS // 16, _reduce, 0)
    pltpu.sync_copy(hist.at[pl.ds(0, _NBINS)], out_hbm.at[wid])


def _fin_body(h_ref, o_ref):
    h = h_ref[...]
    t = jnp.sum(h, axis=0, keepdims=True)
    tot = jnp.sum(t)
    o_ref[...] = jnp.where(tot > 0.0, t / tot, t)


def _tc_finalize(parts):
    return pl.pallas_call(
        _fin_body,
        out_shape=jax.ShapeDtypeStruct((1, _NBINS), jnp.float32),
    )(parts)


def kernel(events):
    ev3 = events.reshape(_NROWS, 128, 4).transpose(0, 2, 1)
    mins, maxs = _sc_minmax(ev3)
    parts = _sc_hist(ev3, mins, maxs)
    flat = _tc_finalize(parts)
    return flat.reshape(2, _VG_T, _VG_H, _VG_W)


# speedup vs baseline: 1.1043x; 1.1043x over previous
import functools

import jax
import jax.numpy as jnp
from jax import lax
from jax.experimental import pallas as pl
from jax.experimental.pallas import tpu as pltpu
from jax.experimental.pallas import tpu_sc as plsc

_VG_W, _VG_H, _VG_T = 32, 24, 5
_XY_SCALE = 0.05
_NBINS = 2 * _VG_T * _VG_H * _VG_W
_NLANE = 16
_NTILES = 32

_N = 4_000_000
_NROWS = _N // 128

_HROWS = 61
_HFULL = 976 // _HROWS

_MROWS = 504
_MCH = 2

_COPIES = 8
_SKEW = _NBINS + 1
_HALLOC = _COPIES * _SKEW

_SC_PARAMS = pltpu.CompilerParams(
    needs_layout_passes=False, use_tc_tiling_on_sc=False)
_sc_mesh = plsc.VectorSubcoreMesh(core_axis_name="c", subcore_axis_name="s")


def _dma_start(src, dst, sem):
    pltpu.make_async_copy(src, dst, sem).start()


def _dma_wait(src, dst, sem):
    pltpu.make_async_copy(src, dst, sem).wait()


@functools.partial(
    pl.kernel,
    mesh=_sc_mesh,
    compiler_params=_SC_PARAMS,
    out_type=(
        jax.ShapeDtypeStruct((_NTILES, 16), jnp.float32),
        jax.ShapeDtypeStruct((_NTILES, 16), jnp.float32),
    ),
    scratch_types=[
        pltpu.VMEM((_MROWS, 1, 128), jnp.float32),
        pltpu.VMEM((_MROWS, 1, 128), jnp.float32),
        pltpu.VMEM((16,), jnp.float32),
        pltpu.VMEM((16,), jnp.float32),
        pltpu.SemaphoreType.DMA,
        pltpu.SemaphoreType.DMA,
    ],
)
def _sc_minmax(ev_hbm, min_hbm, max_hbm, bufa, bufb, minv, maxv, sema, semb):
    wid = lax.axis_index("s") * 2 + lax.axis_index("c")
    s = 976 * wid

    def _st(c):
        return jnp.minimum(s + c * _MROWS, _NROWS - _MROWS)

    _dma_start(ev_hbm.at[pl.ds(_st(0), _MROWS), pl.ds(2, 1), :], bufa, sema)
    _dma_start(ev_hbm.at[pl.ds(_st(1), _MROWS), pl.ds(2, 1), :], bufb, semb)

    def _scan(buf, mn, mx):
        def rbody(j, c):
            mn, mx = c
            for a in range(8):
                t = buf[j, 0, pl.ds(a * 16, 16)]
                mn = jnp.minimum(mn, t)
                mx = jnp.maximum(mx, t)
            return mn, mx

        return lax.fori_loop(0, _MROWS, rbody, (mn, mx))

    def _main(i, c):
        mn, mx = c
        c0 = 2 * i
        _dma_wait(ev_hbm.at[pl.ds(0, _MROWS), pl.ds(2, 1), :], bufa, sema)
        mn, mx = _scan(bufa, mn, mx)

        @pl.when(c0 + 2 < _MCH)
        def _():
            _dma_start(ev_hbm.at[pl.ds(_st(c0 + 2), _MROWS), pl.ds(2, 1), :],
                       bufa, sema)

        _dma_wait(ev_hbm.at[pl.ds(0, _MROWS), pl.ds(2, 1), :], bufb, semb)
        mn, mx = _scan(bufb, mn, mx)

        @pl.when(c0 + 3 < _MCH)
        def _():
            _dma_start(ev_hbm.at[pl.ds(_st(c0 + 3), _MROWS), pl.ds(2, 1), :],
                       bufb, semb)

        return mn, mx

    inf = jnp.full((16,), jnp.inf, jnp.float32)
    mn, mx = lax.fori_loop(0, _MCH // 2, _main, (inf, -inf))
    minv[...] = mn
    maxv[...] = mx
    pltpu.sync_copy(minv, min_hbm.at[wid])
    pltpu.sync_copy(maxv, max_hbm.at[wid])


@functools.partial(
    pl.kernel,
    mesh=_sc_mesh,
    compiler_params=_SC_PARAMS,
    out_type=jax.ShapeDtypeStruct((_NTILES, _NBINS), jnp.float32),
    scratch_types=[
        pltpu.VMEM((_HALLOC,), jnp.float32),
        pltpu.VMEM((_HROWS, 4, 128), jnp.float32),
        pltpu.VMEM((_HROWS, 4, 128), jnp.float32),
        pltpu.VMEM((_NTILES, 16), jnp.float32),
        pltpu.VMEM((_NTILES, 16), jnp.float32),
        pltpu.SemaphoreType.DMA,
        pltpu.SemaphoreType.DMA,
    ],
)
def _sc_hist(ev_hbm, min_hbm, max_hbm, out_hbm,
             hist, bufa, bufb, mnv, mxv, sema, semb):
    wid = lax.axis_index("s") * 2 + lax.axis_index("c")
    s = 976 * wid + jnp.minimum(wid, 18)
    n = jnp.where(wid < 18, 977, 976)

    ii = lax.iota(jnp.int32, 16)
    z16 = jnp.zeros((16,), jnp.float32)
    ones = jnp.ones((16,), jnp.float32)
    lane_offs = [((ii + a) & (_COPIES - 1)) * _SKEW for a in range(8)]

    _dma_start(ev_hbm.at[pl.ds(s, _HROWS), :, :], bufa, sema)
    _dma_start(ev_hbm.at[pl.ds(s + _HROWS, _HROWS), :, :], bufb, semb)

    pltpu.sync_copy(min_hbm, mnv)
    pltpu.sync_copy(max_hbm, mxv)
    mn = mnv[0, :]
    mx = mxv[0, :]
    for r in range(1, _NTILES):
        mn = jnp.minimum(mn, mnv[r, :])
        mx = jnp.maximum(mx, mxv[r, :])
    tmin = jnp.full((16,), jnp.min(mn), jnp.float32)
    tmax = jnp.full((16,), jnp.max(mx), jnp.float32)
    condv = tmax > tmin
    denom = jnp.where(condv, tmax - tmin, 1.0)
    tscl = jnp.where(condv, jnp.float32(_VG_T) / denom, jnp.float32(0.1))
    toff = jnp.where(condv, tmin, 0.0)

    def _zero(i, c):
        b = i * 256
        for k in range(16):
            hist[pl.ds(b + k * 16, 16)] = z16
        return c

    lax.fori_loop(0, _HALLOC // 256, _zero, 0)
    hist[pl.ds(_HALLOC - 16, 16)] = z16

    def _bins(buf, j, a):
        sl = pl.ds(a * 16, 16)
        x = buf[j, 0, sl]
        y = buf[j, 1, sl]
        t = buf[j, 2, sl]
        p = buf[j, 3, sl]
        xv = jnp.clip((x * _XY_SCALE).astype(jnp.int32), 0, _VG_W - 1)
        yv = jnp.clip((y * _XY_SCALE).astype(jnp.int32), 0, _VG_H - 1)
        tv = jnp.clip(((t - toff) * tscl).astype(jnp.int32), 0, _VG_T - 1)
        ch = jnp.where(p > 0.0, 0, _NBINS // 2)
        return ch + tv * (_VG_H * _VG_W) + yv * _VG_W + xv + lane_offs[a]

    def _proc(buf):
        @plsc.parallel_loop(0, _HROWS, unroll=2)
        def rbody(j):
            bs = [_bins(buf, j, a) for a in range(8)]
            for b in bs:
                plsc.addupdate_scatter(hist, [b], ones)

    _TAILW = 977 - _HFULL * _HROWS

    def _main(i, c):
        c0 = 2 * i
        _dma_wait(ev_hbm.at[pl.ds(0, _HROWS), :, :], bufa, sema)
        _proc(bufa)

        @pl.when(c0 + 2 < _HFULL)
        def _():
            _dma_start(ev_hbm.at[pl.ds(s + (c0 + 2) * _HROWS, _HROWS), :, :],
                       bufa, sema)

        _dma_wait(ev_hbm.at[pl.ds(0, _HROWS), :, :], bufb, semb)
        _proc(bufb)

        @pl.when(c0 + 3 < _HFULL)
        def _():
            _dma_start(ev_hbm.at[pl.ds(s + (c0 + 3) * _HROWS, _HROWS), :, :],
                       bufb, semb)

        return c

    lax.fori_loop(0, _HFULL // 2, _main, 0)

    pltpu.sync_copy(ev_hbm.at[pl.ds(s + n - _TAILW, _TAILW), :, :],
                    bufa.at[pl.ds(0, _TAILW), :, :])

    def tbody(r, c):
        keep = (jnp.full((16,), r, jnp.int32) + n) >= 977
        for a in range(8):
            plsc.addupdate_scatter(hist, [_bins(bufa, r, a)], ones, mask=keep)
        return c

    lax.fori_loop(0, _TAILW, tbody, 0)

    def _reduce(j, c):
        o = j * 16
        acc = hist[pl.ds(o, 16)]
        for l in range(1, _COPIES):
            acc = acc + hist[pl.ds(l * _SKEW + o, 16)]
        hist[pl.ds(o, 16)] = acc
        return c

    lax.fori_loop(0, _NBINS // 16, _reduce, 0)
    pltpu.sync_copy(hist.at[pl.ds(0, _NBINS)], out_hbm.at[wid])


def _fin_body(h_ref, o_ref):
    h = h_ref[...]
    t = jnp.sum(h, axis=0, keepdims=True)
    tot = jnp.sum(t)
    o_ref[...] = jnp.where(tot > 0.0, t / tot, t)


def _tc_finalize(parts):
    return pl.pallas_call(
        _fin_body,
        out_shape=jax.ShapeDtypeStruct((1, _NBINS), jnp.float32),
    )(parts)


def kernel(events):
    ev3 = events.reshape(_NROWS, 128, 4).transpose(0, 2, 1)
    mins, maxs = _sc_minmax(ev3)
    parts = _sc_hist(ev3, mins, maxs)
    flat = _tc_finalize(parts)
    return flat.reshape(2, _VG_T, _VG_H, _VG_W)


# speedup vs baseline: 1.1304x; 1.0236x over previous
import functools

import jax
import jax.numpy as jnp
from jax import lax
from jax.experimental import pallas as pl
from jax.experimental.pallas import tpu as pltpu
from jax.experimental.pallas import tpu_sc as plsc

_VG_W, _VG_H, _VG_T = 32, 24, 5
_XY_SCALE = 0.05
_NBINS = 2 * _VG_T * _VG_H * _VG_W
_NLANE = 16
_NTILES = 32

_N = 4_000_000
_NROWS = _N // 128

_HROWS = 61
_HFULL = 976 // _HROWS

_MROWS = 504
_MCH = 2

_COPIES = 8
_SKEW = _NBINS + 1
_HALLOC = _COPIES * _SKEW

_SC_PARAMS = pltpu.CompilerParams(
    needs_layout_passes=False, use_tc_tiling_on_sc=False)
_sc_mesh = plsc.VectorSubcoreMesh(core_axis_name="c", subcore_axis_name="s")


def _dma_start(src, dst, sem):
    pltpu.make_async_copy(src, dst, sem).start()


def _dma_wait(src, dst, sem):
    pltpu.make_async_copy(src, dst, sem).wait()


@functools.partial(
    pl.kernel,
    mesh=_sc_mesh,
    compiler_params=_SC_PARAMS,
    out_type=(
        jax.ShapeDtypeStruct((_NTILES, 16), jnp.float32),
        jax.ShapeDtypeStruct((_NTILES, 16), jnp.float32),
    ),
    scratch_types=[
        pltpu.VMEM((_MROWS, 1, 128), jnp.float32),
        pltpu.VMEM((_MROWS, 1, 128), jnp.float32),
        pltpu.VMEM((16,), jnp.float32),
        pltpu.VMEM((16,), jnp.float32),
        pltpu.SemaphoreType.DMA,
        pltpu.SemaphoreType.DMA,
    ],
)
def _sc_minmax(ev_hbm, min_hbm, max_hbm, bufa, bufb, minv, maxv, sema, semb):
    wid = lax.axis_index("s") * 2 + lax.axis_index("c")
    s = 976 * wid

    def _st(c):
        return jnp.minimum(s + c * _MROWS, _NROWS - _MROWS)

    _dma_start(ev_hbm.at[pl.ds(_st(0), _MROWS), pl.ds(2, 1), :], bufa, sema)
    _dma_start(ev_hbm.at[pl.ds(_st(1), _MROWS), pl.ds(2, 1), :], bufb, semb)

    def _scan(buf, mn, mx):
        def rbody(j, c):
            mn, mx = c
            for a in range(8):
                t = buf[j, 0, pl.ds(a * 16, 16)]
                mn = jnp.minimum(mn, t)
                mx = jnp.maximum(mx, t)
            return mn, mx

        return lax.fori_loop(0, _MROWS, rbody, (mn, mx))

    def _main(i, c):
        mn, mx = c
        c0 = 2 * i
        _dma_wait(ev_hbm.at[pl.ds(0, _MROWS), pl.ds(2, 1), :], bufa, sema)
        mn, mx = _scan(bufa, mn, mx)

        @pl.when(c0 + 2 < _MCH)
        def _():
            _dma_start(ev_hbm.at[pl.ds(_st(c0 + 2), _MROWS), pl.ds(2, 1), :],
                       bufa, sema)

        _dma_wait(ev_hbm.at[pl.ds(0, _MROWS), pl.ds(2, 1), :], bufb, semb)
        mn, mx = _scan(bufb, mn, mx)

        @pl.when(c0 + 3 < _MCH)
        def _():
            _dma_start(ev_hbm.at[pl.ds(_st(c0 + 3), _MROWS), pl.ds(2, 1), :],
                       bufb, semb)

        return mn, mx

    inf = jnp.full((16,), jnp.inf, jnp.float32)
    mn, mx = lax.fori_loop(0, _MCH // 2, _main, (inf, -inf))
    minv[...] = mn
    maxv[...] = mx
    pltpu.sync_copy(minv, min_hbm.at[wid])
    pltpu.sync_copy(maxv, max_hbm.at[wid])


@functools.partial(
    pl.kernel,
    mesh=_sc_mesh,
    compiler_params=_SC_PARAMS,
    out_type=jax.ShapeDtypeStruct((_NTILES * _NBINS,), jnp.float32),
    scratch_types=[
        pltpu.VMEM((_HALLOC,), jnp.float32),
        pltpu.VMEM((_HROWS, 4, 128), jnp.float32),
        pltpu.VMEM((_HROWS, 4, 128), jnp.float32),
        pltpu.VMEM((_NTILES, 16), jnp.float32),
        pltpu.VMEM((_NTILES, 16), jnp.float32),
        pltpu.SemaphoreType.DMA,
        pltpu.SemaphoreType.DMA,
    ],
)
def _sc_hist(ev_hbm, min_hbm, max_hbm, out_hbm,
             hist, bufa, bufb, mnv, mxv, sema, semb):
    wid = lax.axis_index("s") * 2 + lax.axis_index("c")
    s = 976 * wid + jnp.minimum(wid, 18)
    n = jnp.where(wid < 18, 977, 976)

    ii = lax.iota(jnp.int32, 16)
    z16 = jnp.zeros((16,), jnp.float32)
    ones = jnp.ones((16,), jnp.float32)
    lane_offs = [((ii + a) & (_COPIES - 1)) * _SKEW for a in range(8)]

    _dma_start(ev_hbm.at[pl.ds(s, _HROWS), :, :], bufa, sema)
    _dma_start(ev_hbm.at[pl.ds(s + _HROWS, _HROWS), :, :], bufb, semb)

    pltpu.sync_copy(min_hbm, mnv)
    pltpu.sync_copy(max_hbm, mxv)
    mn = mnv[0, :]
    mx = mxv[0, :]
    for r in range(1, _NTILES):
        mn = jnp.minimum(mn, mnv[r, :])
        mx = jnp.maximum(mx, mxv[r, :])
    tmin = jnp.full((16,), jnp.min(mn), jnp.float32)
    tmax = jnp.full((16,), jnp.max(mx), jnp.float32)
    condv = tmax > tmin
    denom = jnp.where(condv, tmax - tmin, 1.0)
    tscl = jnp.where(condv, jnp.float32(_VG_T) / denom, jnp.float32(0.1))
    toff = jnp.where(condv, tmin, 0.0)

    def _zero(i, c):
        b = i * 256
        for k in range(16):
            hist[pl.ds(b + k * 16, 16)] = z16
        return c

    lax.fori_loop(0, _HALLOC // 256, _zero, 0)
    hist[pl.ds(_HALLOC - 16, 16)] = z16

    def _bins(buf, j, a):
        sl = pl.ds(a * 16, 16)
        x = buf[j, 0, sl]
        y = buf[j, 1, sl]
        t = buf[j, 2, sl]
        p = buf[j, 3, sl]
        xv = jnp.clip((x * _XY_SCALE).astype(jnp.int32), 0, _VG_W - 1)
        yv = jnp.clip((y * _XY_SCALE).astype(jnp.int32), 0, _VG_H - 1)
        tv = jnp.clip(((t - toff) * tscl).astype(jnp.int32), 0, _VG_T - 1)
        ch = jnp.where(p > 0.0, 0, _NBINS // 2)
        return ch + tv * (_VG_H * _VG_W) + yv * _VG_W + xv + lane_offs[a]

    def _proc(buf):
        @plsc.parallel_loop(0, _HROWS, unroll=2)
        def rbody(j):
            bs = [_bins(buf, j, a) for a in range(8)]
            for b in bs:
                plsc.addupdate_scatter(hist, [b], ones)

    _TAILW = 977 - _HFULL * _HROWS

    def _main(i, c):
        c0 = 2 * i
        _dma_wait(ev_hbm.at[pl.ds(0, _HROWS), :, :], bufa, sema)
        _proc(bufa)

        @pl.when(c0 + 2 < _HFULL)
        def _():
            _dma_start(ev_hbm.at[pl.ds(s + (c0 + 2) * _HROWS, _HROWS), :, :],
                       bufa, sema)

        _dma_wait(ev_hbm.at[pl.ds(0, _HROWS), :, :], bufb, semb)
        _proc(bufb)

        @pl.when(c0 + 3 < _HFULL)
        def _():
            _dma_start(ev_hbm.at[pl.ds(s + (c0 + 3) * _HROWS, _HROWS), :, :],
                       bufb, semb)

        return c

    lax.fori_loop(0, _HFULL // 2, _main, 0)

    pltpu.sync_copy(ev_hbm.at[pl.ds(s + n - _TAILW, _TAILW), :, :],
                    bufa.at[pl.ds(0, _TAILW), :, :])

    def tbody(r, c):
        keep = (jnp.full((16,), r, jnp.int32) + n) >= 977
        for a in range(8):
            plsc.addupdate_scatter(hist, [_bins(bufa, r, a)], ones, mask=keep)
        return c

    lax.fori_loop(0, _TAILW, tbody, 0)

    def _reduce(j, c):
        o = j * 16
        acc = hist[pl.ds(o, 16)]
        for l in range(1, _COPIES):
            acc = acc + hist[pl.ds(l * _SKEW + o, 16)]
        hist[pl.ds(o, 16)] = acc
        return c

    lax.fori_loop(0, _NBINS // 16, _reduce, 0)
    pltpu.sync_copy(hist.at[pl.ds(0, _NBINS)],
                    out_hbm.at[pl.ds(wid * _NBINS, _NBINS)])


def _fin_body(h_ref, o_ref):
    h = h_ref[...].reshape(_NTILES, _NBINS // 128, 128)
    t = jnp.sum(h, axis=0)
    tot = jnp.sum(t)
    o_ref[...] = jnp.where(tot > 0.0, t / tot, t)


def _tc_finalize(parts):
    return pl.pallas_call(
        _fin_body,
        out_shape=jax.ShapeDtypeStruct((_NBINS // 128, 128), jnp.float32),
    )(parts.reshape(_NTILES * _NBINS // 128, 128))


def kernel(events):
    ev3 = events.reshape(_NROWS, 128, 4).transpose(0, 2, 1)
    mins, maxs = _sc_minmax(ev3)
    parts = _sc_hist(ev3, mins, maxs)
    flat = _tc_finalize(parts)
    return flat.reshape(2, _VG_T, _VG_H, _VG_W)
